# Initial kernel scaffold; baseline (speedup 1.0000x reference)
import jax
import jax.numpy as jnp
from jax.experimental import pallas as pl


def kernel(part):
    raise NotImplementedError("write your pallas kernel here")


# speedup vs baseline: 1.0645x; 1.0645x over previous
import jax
import jax.numpy as jnp
from jax.experimental import pallas as pl

_PIDX = (4, 2, 3, 3, 0, 1, 1, 4, 4, 5, 6, 7, 7, 8, 9, 9)
_D = 256


def _body(in_ref, out_ref):
    for j, p in enumerate(_PIDX):
        out_ref[:, j * _D:(j + 1) * _D] = in_ref[:, p * _D:(p + 1) * _D]


def kernel(part):
    n, npart, d = part.shape
    x = part.reshape(n, npart * d)
    bn = 256
    out = pl.pallas_call(
        _body,
        grid=(n // bn,),
        in_specs=[pl.BlockSpec((bn, npart * d), lambda i: (i, 0))],
        out_specs=pl.BlockSpec((bn, 16 * d), lambda i: (i, 0)),
        out_shape=jax.ShapeDtypeStruct((n, 16 * d), jnp.float32),
    )(x)
    return out.reshape(n, 16, d)
